# baseline (speedup 1.0000x reference)
import functools

import jax
import jax.numpy as jnp
from jax import lax
from jax.experimental import pallas as pl
from jax.experimental.pallas import tpu as pltpu
from jax.experimental.pallas import tpu_sc as plsc

B, L, D = 8, 512, 256
ML = 3584
NW = 32
WPR = NW // B
POS_PW = ML // WPR
CHUNK = 128
NCH = POS_PW // CHUNK
ZROW = B * L


NB = 2


def _lr_body(xpad_hbm, dur_hbm, out_hbm, mel_hbm,
             d_v, s_v, ix_v, mel_v, bufs, gsems, wsems):
    cid = lax.axis_index("c")
    sid = lax.axis_index("s")
    wid = cid * 16 + sid
    row = wid // WPR
    quarter = wid % WPR

    pltpu.sync_copy(dur_hbm.at[row], d_v)

    lane = lax.iota(jnp.int32, 16)

    def _zero(j, _):
        s_v[pl.ds(j * 16, 16)] = jnp.zeros((16,), jnp.int32)
        return 0
    lax.fori_loop(0, ML // 16, _zero, 0)

    def _scan(j, carry):
        d = d_v[pl.ds(j * 16, 16)]
        cs = plsc.cumsum(d) + carry
        start = cs - d
        tok = j * 16 + lane
        m = (d > 0) & (start < ML)
        plsc.store_scatter(s_v, [start], tok, mask=m)
        return jnp.max(cs)
    total = lax.fori_loop(0, L // 16, _scan, jnp.int32(0))

    rbase = row * L
    zrow = ZROW

    def _cmax(j, carry):
        v = s_v[pl.ds(j * 16, 16)]
        cm = jnp.maximum(plsc.cummax(v), carry)
        pos = j * 16 + lane
        src = jnp.where(pos < total, rbase + cm, zrow)
        ix_v[pl.ds(j * 16, 16)] = src
        return jnp.max(cm)
    lax.fori_loop(0, ML // 16, _cmax, jnp.int32(0))

    @pl.when(quarter == 0)
    def _():
        mel_v[...] = jnp.full((16,), total, jnp.int32)
        pltpu.sync_copy(mel_v, mel_hbm.at[row])

    qoff = quarter * POS_PW
    obase = row * ML + qoff
    gcps = [None] * NCH
    wcps = [None] * NCH
    for c in range(NCH):
        k = c % NB
        if c >= NB:
            wcps[c - NB].wait()
        gcps[c] = pltpu.async_copy(
            xpad_hbm.at[ix_v.at[pl.ds(qoff + c * CHUNK, CHUNK)]],
            bufs[k], gsems[k])
        if c > 0:
            gcps[c - 1].wait()
            wcps[c - 1] = pltpu.async_copy(
                bufs[(c - 1) % NB],
                out_hbm.at[pl.ds(obase + (c - 1) * CHUNK, CHUNK)],
                wsems[(c - 1) % NB])
    gcps[NCH - 1].wait()
    wcps[NCH - 1] = pltpu.async_copy(
        bufs[(NCH - 1) % NB],
        out_hbm.at[pl.ds(obase + (NCH - 1) * CHUNK, CHUNK)],
        wsems[(NCH - 1) % NB])
    for c in range(max(0, NCH - NB), NCH):
        wcps[c].wait()


_lr_call = functools.partial(
    pl.kernel,
    out_type=[
        jax.ShapeDtypeStruct((B * ML, D), jnp.float32),
        jax.ShapeDtypeStruct((B, 16), jnp.int32),
    ],
    mesh=plsc.VectorSubcoreMesh(core_axis_name="c", subcore_axis_name="s"),
    compiler_params=pltpu.CompilerParams(
        needs_layout_passes=False, use_tc_tiling_on_sc=False),
    scratch_types=[
        pltpu.VMEM((L,), jnp.int32),
        pltpu.VMEM((ML,), jnp.int32),
        pltpu.VMEM((ML,), jnp.int32),
        pltpu.VMEM((16,), jnp.int32),
        [pltpu.VMEM((CHUNK, D), jnp.float32) for _ in range(NB)],
        [pltpu.SemaphoreType.DMA for _ in range(NB)],
        [pltpu.SemaphoreType.DMA for _ in range(NB)],
    ],
)(_lr_body)


def kernel(x, duration, max_len):
    xpad = jnp.concatenate(
        [x.reshape(B * L, D), jnp.zeros((8, D), x.dtype)], axis=0)
    out_flat, mel = _lr_call(xpad, duration)
    return out_flat.reshape(B, ML, D), mel[:, 0]


# speedup vs baseline: 4.6600x; 4.6600x over previous
import functools

import jax
import jax.numpy as jnp
from jax import lax
from jax.experimental import pallas as pl
from jax.experimental.pallas import tpu as pltpu
from jax.experimental.pallas import tpu_sc as plsc

B, L, D = 8, 512, 256
ML = 3584
NW = 32
WPR = NW // B
POS_PW = ML // WPR
CHUNK = 112
NCH = POS_PW // CHUNK
SRCW = CHUNK + 1
XPAD = B * L + 128
NB = 2


def _lr_body(xpad_hbm, dur_hbm, out_hbm, mel_hbm,
             d_v, s_v, ix_v, mel_v, sbufs, obufs, gsems, wsems):
    cid = lax.axis_index("c")
    sid = lax.axis_index("s")
    wid = cid * 16 + sid
    row = wid // WPR
    quarter = wid % WPR

    pltpu.sync_copy(dur_hbm.at[row], d_v)

    lane = lax.iota(jnp.int32, 16)

    def _zero(j, _):
        s_v[pl.ds(j * 16, 16)] = jnp.zeros((16,), jnp.int32)
        return 0
    lax.fori_loop(0, ML // 16, _zero, 0)

    def _scan(j, carry):
        d = d_v[pl.ds(j * 16, 16)]
        cs = plsc.cumsum(d) + carry
        start = cs - d
        tok = j * 16 + lane
        m = (d > 0) & (start < ML)
        plsc.store_scatter(s_v, [start], tok, mask=m)
        return jnp.max(cs)
    total = lax.fori_loop(0, L // 16, _scan, jnp.int32(0))

    def _cmax(j, carry):
        v = s_v[pl.ds(j * 16, 16)]
        cm = jnp.maximum(plsc.cummax(v), carry)
        ix_v[pl.ds(j * 16, 16)] = cm
        return jnp.max(cm)
    lax.fori_loop(0, ML // 16, _cmax, jnp.int32(0))

    @pl.when(quarter == 0)
    def _():
        mel_v[...] = jnp.full((16,), total, jnp.int32)
        pltpu.sync_copy(mel_v, mel_hbm.at[row])

    qoff = quarter * POS_PW
    obase = row * ML + qoff
    zeros16 = jnp.zeros((16,), jnp.float32)

    def _stage(c, k):
        lo = ix_v[pl.ds(qoff + c * CHUNK, 16)][0]
        return lo, pltpu.async_copy(
            xpad_hbm.at[pl.ds(row * L + lo, SRCW)], sbufs[k], gsems[k])

    def _expand(c, k, lo):
        sbuf, obuf = sbufs[k], obufs[k]
        base_p = qoff + c * CHUNK
        nvalid = jnp.clip(total - base_p, 0, CHUNK)

        def _copy_row(r, _):
            t = ix_v[pl.ds(base_p + r, 16)][0]
            off = t - lo

            @pl.when(off < SRCW)
            def _():
                for j in range(D // 16):
                    obuf[r, pl.ds(j * 16, 16)] = sbuf[off, pl.ds(j * 16, 16)]

            @pl.when(off >= SRCW)
            def _():
                pltpu.sync_copy(xpad_hbm.at[pl.ds(row * L + t, 1)],
                                obuf.at[pl.ds(r, 1)])
            return 0
        lax.fori_loop(0, nvalid, _copy_row, 0)

        def _zero_row(r, _):
            for j in range(D // 16):
                obuf[r, pl.ds(j * 16, 16)] = zeros16
            return 0
        lax.fori_loop(nvalid, CHUNK, _zero_row, 0)

    wcps = [None] * NCH
    lo0, scp0 = _stage(0, 0)
    los = {0: lo0}
    scps = {0: scp0}
    for c in range(NCH):
        k = c % NB
        kn = (c + 1) % NB
        if c + 1 < NCH:
            if c + 1 >= NB:
                wcps[c + 1 - NB].wait()
            los[c + 1], scps[c + 1] = _stage(c + 1, kn)
        scps[c].wait()
        _expand(c, k, los[c])
        wcps[c] = pltpu.async_copy(
            obufs[k], out_hbm.at[pl.ds(obase + c * CHUNK, CHUNK)], wsems[k])
    for c in range(NCH - NB, NCH):
        wcps[c].wait()


_lr_call = functools.partial(
    pl.kernel,
    out_type=[
        jax.ShapeDtypeStruct((B * ML, D), jnp.float32),
        jax.ShapeDtypeStruct((B, 16), jnp.int32),
    ],
    mesh=plsc.VectorSubcoreMesh(core_axis_name="c", subcore_axis_name="s"),
    compiler_params=pltpu.CompilerParams(
        needs_layout_passes=False, use_tc_tiling_on_sc=False),
    scratch_types=[
        pltpu.VMEM((L,), jnp.int32),
        pltpu.VMEM((ML,), jnp.int32),
        pltpu.VMEM((ML + 16,), jnp.int32),
        pltpu.VMEM((16,), jnp.int32),
        [pltpu.VMEM((SRCW, D), jnp.float32) for _ in range(NB)],
        [pltpu.VMEM((CHUNK, D), jnp.float32) for _ in range(NB)],
        [pltpu.SemaphoreType.DMA for _ in range(NB)],
        [pltpu.SemaphoreType.DMA for _ in range(NB)],
    ],
)(_lr_body)


def kernel(x, duration, max_len):
    xpad = jnp.concatenate(
        [x.reshape(B * L, D), jnp.zeros((XPAD - B * L, D), x.dtype)], axis=0)
    out_flat, mel = _lr_call(xpad, duration)
    return out_flat.reshape(B, ML, D), mel[:, 0]


# speedup vs baseline: 4.6746x; 1.0031x over previous
import functools

import jax
import jax.numpy as jnp
from jax import lax
from jax.experimental import pallas as pl
from jax.experimental.pallas import tpu as pltpu
from jax.experimental.pallas import tpu_sc as plsc

B, L, D = 8, 512, 256
ML = 3584
NW = 32
WPR = NW // B
POS_PW = ML // WPR
CHUNK = 112
NCH = POS_PW // CHUNK
SRCW = CHUNK + 1
NB = 2


def _lr_body(x_hbm, dur_hbm, out_hbm, mel_hbm,
             d_v, s_v, ix_v, mel_v, sbufs, obufs, gsems, wsems):
    cid = lax.axis_index("c")
    sid = lax.axis_index("s")
    wid = cid * 16 + sid
    row = wid // WPR
    quarter = wid % WPR

    pltpu.sync_copy(dur_hbm.at[row], d_v)

    lane = lax.iota(jnp.int32, 16)

    def _zero(j, _):
        s_v[pl.ds(j * 16, 16)] = jnp.zeros((16,), jnp.int32)
        return 0
    lax.fori_loop(0, ML // 16, _zero, 0)

    def _scan(j, carry):
        d = d_v[pl.ds(j * 16, 16)]
        cs = plsc.cumsum(d) + carry
        start = cs - d
        tok = j * 16 + lane
        m = (d > 0) & (start < ML)
        plsc.store_scatter(s_v, [start], tok, mask=m)
        return jnp.max(cs)
    total = lax.fori_loop(0, L // 16, _scan, jnp.int32(0))

    def _cmax(j, carry):
        v = s_v[pl.ds(j * 16, 16)]
        cm = jnp.maximum(plsc.cummax(v), carry)
        ix_v[pl.ds(j * 16, 16)] = cm
        return jnp.max(cm)
    lax.fori_loop(0, ML // 16, _cmax, jnp.int32(0))

    @pl.when(quarter == 0)
    def _():
        mel_v[...] = jnp.full((16,), total, jnp.int32)
        pltpu.sync_copy(mel_v, mel_hbm.at[row])

    qoff = quarter * POS_PW
    obase = row * ML + qoff
    zeros16 = jnp.zeros((16,), jnp.float32)

    def _stage(c, k):
        lo = ix_v[pl.ds(qoff + c * CHUNK, 16)][0]
        start = jnp.minimum(row * L + lo, B * L - SRCW)
        return start, pltpu.async_copy(
            x_hbm.at[pl.ds(start, SRCW)], sbufs[k], gsems[k])

    def _expand(c, k, start):
        sbuf, obuf = sbufs[k], obufs[k]
        base_p = qoff + c * CHUNK
        nvalid = jnp.clip(total - base_p, 0, CHUNK)

        def _copy_row(r, _):
            t = ix_v[pl.ds(base_p + r, 16)][0]
            off = row * L + t - start

            @pl.when(off < SRCW)
            def _():
                for j in range(D // 16):
                    obuf[r, pl.ds(j * 16, 16)] = sbuf[off, pl.ds(j * 16, 16)]

            @pl.when(off >= SRCW)
            def _():
                pltpu.sync_copy(x_hbm.at[pl.ds(row * L + t, 1)],
                                obuf.at[pl.ds(r, 1)])
            return 0
        lax.fori_loop(0, nvalid, _copy_row, 0)

        def _zero_row(r, _):
            for j in range(D // 16):
                obuf[r, pl.ds(j * 16, 16)] = zeros16
            return 0
        lax.fori_loop(nvalid, CHUNK, _zero_row, 0)

    wcps = [None] * NCH
    lo0, scp0 = _stage(0, 0)
    los = {0: lo0}
    scps = {0: scp0}
    for c in range(NCH):
        k = c % NB
        kn = (c + 1) % NB
        if c + 1 < NCH:
            if c + 1 >= NB:
                wcps[c + 1 - NB].wait()
            los[c + 1], scps[c + 1] = _stage(c + 1, kn)
        scps[c].wait()
        _expand(c, k, los[c])
        wcps[c] = pltpu.async_copy(
            obufs[k], out_hbm.at[pl.ds(obase + c * CHUNK, CHUNK)], wsems[k])
    for c in range(NCH - NB, NCH):
        wcps[c].wait()


_lr_call = functools.partial(
    pl.kernel,
    out_type=[
        jax.ShapeDtypeStruct((B * ML, D), jnp.float32),
        jax.ShapeDtypeStruct((B, 16), jnp.int32),
    ],
    mesh=plsc.VectorSubcoreMesh(core_axis_name="c", subcore_axis_name="s"),
    compiler_params=pltpu.CompilerParams(
        needs_layout_passes=False, use_tc_tiling_on_sc=False),
    scratch_types=[
        pltpu.VMEM((L,), jnp.int32),
        pltpu.VMEM((ML,), jnp.int32),
        pltpu.VMEM((ML + 16,), jnp.int32),
        pltpu.VMEM((16,), jnp.int32),
        [pltpu.VMEM((SRCW, D), jnp.float32) for _ in range(NB)],
        [pltpu.VMEM((CHUNK, D), jnp.float32) for _ in range(NB)],
        [pltpu.SemaphoreType.DMA for _ in range(NB)],
        [pltpu.SemaphoreType.DMA for _ in range(NB)],
    ],
)(_lr_body)


def kernel(x, duration, max_len):
    out_flat, mel = _lr_call(x.reshape(B * L, D), duration)
    return out_flat.reshape(B, ML, D), mel[:, 0]


# speedup vs baseline: 4.7888x; 1.0244x over previous
import functools

import jax
import jax.numpy as jnp
from jax import lax
from jax.experimental import pallas as pl
from jax.experimental.pallas import tpu as pltpu
from jax.experimental.pallas import tpu_sc as plsc

B, L, D = 8, 512, 256
ML = 3584
NW = 32
WPR = NW // B
POS_PW = ML // WPR
CHUNK = 112
NCH = POS_PW // CHUNK
SRCW = CHUNK + 1
NB = 2


def _lr_body(x_hbm, dur_hbm, out_hbm, mel_hbm,
             d_v, s_v, ix_v, mel_v, sbufs, obufs, gsems, wsems):
    cid = lax.axis_index("c")
    sid = lax.axis_index("s")
    wid = cid * 16 + sid
    row = wid // WPR
    quarter = wid % WPR

    pltpu.sync_copy(dur_hbm.at[row], d_v)

    lane = lax.iota(jnp.int32, 16)

    qoff = quarter * POS_PW

    def _zero(j, _):
        s_v[pl.ds(qoff + j * 16, 16)] = jnp.zeros((16,), jnp.int32)
        return 0
    lax.fori_loop(0, POS_PW // 16, _zero, 0)

    def _scan(j, carry):
        tot, c0 = carry
        d = d_v[pl.ds(j * 16, 16)]
        cs = plsc.cumsum(d) + tot
        start = cs - d
        tok = j * 16 + lane
        m = (d > 0) & (start < ML)
        plsc.store_scatter(s_v, [start], tok, mask=m)
        c0 = c0 + plsc.all_reduce_population_count(cs <= qoff)[0]
        return jnp.max(cs), c0
    total, c0 = lax.fori_loop(0, L // 16, _scan,
                              (jnp.int32(0), jnp.int32(0)))

    def _cmax(j, carry):
        v = s_v[pl.ds(qoff + j * 16, 16)]
        cm = jnp.maximum(plsc.cummax(v), carry)
        ix_v[pl.ds(qoff + j * 16, 16)] = cm
        return jnp.max(cm)
    lax.fori_loop(0, POS_PW // 16, _cmax, c0)

    @pl.when(quarter == 0)
    def _():
        mel_v[...] = jnp.full((16,), total, jnp.int32)
        pltpu.sync_copy(mel_v, mel_hbm.at[row])

    obase = row * ML + qoff
    zeros16 = jnp.zeros((16,), jnp.float32)

    def _stage(c, k):
        lo = ix_v[pl.ds(qoff + c * CHUNK, 16)][0]
        start = jnp.minimum(row * L + lo, B * L - SRCW)
        return start, pltpu.async_copy(
            x_hbm.at[pl.ds(start, SRCW)], sbufs[k], gsems[k])

    def _expand(c, k, start):
        sbuf, obuf = sbufs[k], obufs[k]
        base_p = qoff + c * CHUNK
        nvalid = jnp.clip(total - base_p, 0, CHUNK)

        def _copy_row(r, _):
            t = ix_v[pl.ds(base_p + r, 16)][0]
            off = row * L + t - start

            @pl.when(off < SRCW)
            def _():
                for j in range(D // 16):
                    obuf[r, pl.ds(j * 16, 16)] = sbuf[off, pl.ds(j * 16, 16)]

            @pl.when(off >= SRCW)
            def _():
                pltpu.sync_copy(x_hbm.at[pl.ds(row * L + t, 1)],
                                obuf.at[pl.ds(r, 1)])
            return 0
        lax.fori_loop(0, nvalid, _copy_row, 0)

        def _zero_row(r, _):
            for j in range(D // 16):
                obuf[r, pl.ds(j * 16, 16)] = zeros16
            return 0
        lax.fori_loop(nvalid, CHUNK, _zero_row, 0)

    wcps = [None] * NCH
    lo0, scp0 = _stage(0, 0)
    los = {0: lo0}
    scps = {0: scp0}
    for c in range(NCH):
        k = c % NB
        kn = (c + 1) % NB
        if c + 1 < NCH:
            if c + 1 >= NB:
                wcps[c + 1 - NB].wait()
            los[c + 1], scps[c + 1] = _stage(c + 1, kn)
        scps[c].wait()
        _expand(c, k, los[c])
        wcps[c] = pltpu.async_copy(
            obufs[k], out_hbm.at[pl.ds(obase + c * CHUNK, CHUNK)], wsems[k])
    for c in range(NCH - NB, NCH):
        wcps[c].wait()


_lr_call = functools.partial(
    pl.kernel,
    out_type=[
        jax.ShapeDtypeStruct((B * ML, D), jnp.float32),
        jax.ShapeDtypeStruct((B, 16), jnp.int32),
    ],
    mesh=plsc.VectorSubcoreMesh(core_axis_name="c", subcore_axis_name="s"),
    compiler_params=pltpu.CompilerParams(
        needs_layout_passes=False, use_tc_tiling_on_sc=False),
    scratch_types=[
        pltpu.VMEM((L,), jnp.int32),
        pltpu.VMEM((ML,), jnp.int32),
        pltpu.VMEM((ML + 16,), jnp.int32),
        pltpu.VMEM((16,), jnp.int32),
        [pltpu.VMEM((SRCW, D), jnp.float32) for _ in range(NB)],
        [pltpu.VMEM((CHUNK, D), jnp.float32) for _ in range(NB)],
        [pltpu.SemaphoreType.DMA for _ in range(NB)],
        [pltpu.SemaphoreType.DMA for _ in range(NB)],
    ],
)(_lr_body)


def kernel(x, duration, max_len):
    out_flat, mel = _lr_call(x.reshape(B * L, D), duration)
    return out_flat.reshape(B, ML, D), mel[:, 0]


# speedup vs baseline: 4.8741x; 1.0178x over previous
import functools

import jax
import jax.numpy as jnp
from jax import lax
from jax.experimental import pallas as pl
from jax.experimental.pallas import tpu as pltpu
from jax.experimental.pallas import tpu_sc as plsc

B, L, D = 8, 512, 256
ML = 3584
NW = 32
WPR = NW // B
POS_PW = ML // WPR
CHUNK = 112
NCH = POS_PW // CHUNK
SRCW = 56
NB = 2


def _lr_body(x_hbm, dur_hbm, out_hbm, mel_hbm,
             d_v, s_v, ix_v, mel_v, sbufs, obufs, zbuf, gsems, wsems):
    cid = lax.axis_index("c")
    sid = lax.axis_index("s")
    wid = cid * 16 + sid
    row = wid // WPR
    quarter = wid % WPR

    pltpu.sync_copy(dur_hbm.at[row], d_v)

    lane = lax.iota(jnp.int32, 16)

    qoff = quarter * POS_PW

    def _zero(j, _):
        s_v[pl.ds(qoff + j * 16, 16)] = jnp.zeros((16,), jnp.int32)
        return 0
    lax.fori_loop(0, POS_PW // 16, _zero, 0)

    def _scan(j, carry):
        tot, c0 = carry
        d = d_v[pl.ds(j * 16, 16)]
        cs = plsc.cumsum(d) + tot
        start = cs - d
        tok = j * 16 + lane
        m = (d > 0) & (start < ML)
        plsc.store_scatter(s_v, [start], tok, mask=m)
        c0 = c0 + plsc.all_reduce_population_count(cs <= qoff)[0]
        return jnp.max(cs), c0
    total, c0 = lax.fori_loop(0, L // 16, _scan,
                              (jnp.int32(0), jnp.int32(0)))

    def _cmax(j, carry):
        v = s_v[pl.ds(qoff + j * 16, 16)]
        cm = jnp.maximum(plsc.cummax(v), carry)
        ix_v[pl.ds(qoff + j * 16, 16)] = cm
        return jnp.max(cm)
    lax.fori_loop(0, POS_PW // 16, _cmax, c0)

    @pl.when(quarter == 0)
    def _():
        mel_v[...] = jnp.full((16,), total, jnp.int32)
        pltpu.sync_copy(mel_v, mel_hbm.at[row])

    obase = row * ML + qoff
    zeros16 = jnp.zeros((16,), jnp.float32)

    nvalids = [jnp.clip(total - (qoff + c * CHUNK), 0, CHUNK)
               for c in range(NCH)]

    def _window(c):
        lo = ix_v[pl.ds(qoff + c * CHUNK, 16)][0]
        return jnp.minimum(row * L + lo, B * L - SRCW)

    def _stage(c, start):
        @pl.when(nvalids[c] > 0)
        def _():
            pltpu.async_copy(x_hbm.at[pl.ds(start, SRCW)],
                             sbufs[c % NB], gsems[c % NB])

    def _stage_wait(c, start):
        @pl.when(nvalids[c] > 0)
        def _():
            pltpu.make_async_copy(x_hbm.at[pl.ds(start, SRCW)],
                                  sbufs[c % NB], gsems[c % NB]).wait()

    def _expand(c, k, start):
        sbuf, obuf = sbufs[k], obufs[k]
        base_p = qoff + c * CHUNK
        nvalid = nvalids[c]

        def _copy_row(r, _):
            t = ix_v[pl.ds(base_p + r, 16)][0]
            off = row * L + t - start

            @pl.when(off < SRCW)
            def _():
                for j in range(D // 16):
                    obuf[r, pl.ds(j * 16, 16)] = sbuf[off, pl.ds(j * 16, 16)]

            @pl.when(off >= SRCW)
            def _():
                pltpu.sync_copy(x_hbm.at[pl.ds(row * L + t, 1)],
                                obuf.at[pl.ds(r, 1)])
            return 0
        lax.fori_loop(0, nvalid, _copy_row, 0)

        def _zero_row(r, _):
            for j in range(D // 16):
                obuf[r, pl.ds(j * 16, 16)] = zeros16
            return 0
        lax.fori_loop(nvalid, CHUNK, _zero_row, 0)

    starts = [None] * NCH
    starts[0] = _window(0)
    _stage(0, starts[0])

    def _zfill(r, _):
        for j in range(D // 16):
            zbuf[r, pl.ds(j * 16, 16)] = zeros16
        return 0
    lax.fori_loop(0, CHUNK, _zfill, 0)

    for c in range(NCH):
        k = c % NB
        nv = nvalids[c]
        if c + 1 < NCH:
            if c + 1 >= NB:
                pltpu.make_async_copy(
                    obufs[(c + 1) % NB],
                    out_hbm.at[pl.ds(obase + (c + 1 - NB) * CHUNK, CHUNK)],
                    wsems[(c + 1) % NB]).wait()
            starts[c + 1] = _window(c + 1)
            _stage(c + 1, starts[c + 1])
        _stage_wait(c, starts[c])

        @pl.when(nv > 0)
        def _(c=c, k=k):
            _expand(c, k, starts[c])
            pltpu.async_copy(
                obufs[k], out_hbm.at[pl.ds(obase + c * CHUNK, CHUNK)],
                wsems[k])

        @pl.when(nv == 0)
        def _(c=c, k=k):
            pltpu.async_copy(
                zbuf, out_hbm.at[pl.ds(obase + c * CHUNK, CHUNK)], wsems[k])
    for c in range(NCH - NB, NCH):
        pltpu.make_async_copy(
            obufs[c % NB],
            out_hbm.at[pl.ds(obase + c * CHUNK, CHUNK)],
            wsems[c % NB]).wait()


_lr_call = functools.partial(
    pl.kernel,
    out_type=[
        jax.ShapeDtypeStruct((B * ML, D), jnp.float32),
        jax.ShapeDtypeStruct((B, 16), jnp.int32),
    ],
    mesh=plsc.VectorSubcoreMesh(core_axis_name="c", subcore_axis_name="s"),
    compiler_params=pltpu.CompilerParams(
        needs_layout_passes=False, use_tc_tiling_on_sc=False),
    scratch_types=[
        pltpu.VMEM((L,), jnp.int32),
        pltpu.VMEM((ML,), jnp.int32),
        pltpu.VMEM((ML + 16,), jnp.int32),
        pltpu.VMEM((16,), jnp.int32),
        [pltpu.VMEM((SRCW, D), jnp.float32) for _ in range(NB)],
        [pltpu.VMEM((CHUNK, D), jnp.float32) for _ in range(NB)],
        pltpu.VMEM((CHUNK, D), jnp.float32),
        [pltpu.SemaphoreType.DMA for _ in range(NB)],
        [pltpu.SemaphoreType.DMA for _ in range(NB)],
    ],
)(_lr_body)


def kernel(x, duration, max_len):
    out_flat, mel = _lr_call(x.reshape(B * L, D), duration)
    return out_flat.reshape(B, ML, D), mel[:, 0]


# speedup vs baseline: 6.0758x; 1.2465x over previous
import functools

import jax
import jax.numpy as jnp
from jax import lax
from jax.experimental import pallas as pl
from jax.experimental.pallas import tpu as pltpu
from jax.experimental.pallas import tpu_sc as plsc

B, L, D = 8, 512, 256
ML = 3584
NW = 32
WPR = NW // B
POS_PW = ML // WPR
CHUNK = 112
NCH = POS_PW // CHUNK
SRCG = 8
NB = 2


def _lr_body(x_hbm, dur_hbm, out_hbm, mel_hbm,
             d_v, s_v, ix_v, mel_v, sbufs, obufs, zbuf, fbuf,
             gsems, wsems, fsem):
    cid = lax.axis_index("c")
    sid = lax.axis_index("s")
    wid = cid * 16 + sid
    row = wid // WPR
    quarter = wid % WPR

    pltpu.sync_copy(dur_hbm.at[row], d_v)

    lane = lax.iota(jnp.int32, 16)
    qoff = quarter * POS_PW

    def _zero(j, _):
        s_v[pl.ds(qoff + j * 16, 16)] = jnp.zeros((16,), jnp.int32)
        return 0
    lax.fori_loop(0, POS_PW // 16, _zero, 0)

    def _scan(j, carry):
        tot, c0 = carry
        d = d_v[pl.ds(j * 16, 16)]
        cs = plsc.cumsum(d) + tot
        start = cs - d
        tok = j * 16 + lane
        m = (d > 0) & (start < ML)
        plsc.store_scatter(s_v, [start], tok, mask=m)
        c0 = c0 + plsc.all_reduce_population_count(cs <= qoff)[0]
        return jnp.max(cs), c0
    total, c0 = lax.fori_loop(0, L // 16, _scan,
                              (jnp.int32(0), jnp.int32(0)))

    def _cmax(j, carry):
        v = s_v[pl.ds(qoff + j * 16, 16)]
        cm = jnp.maximum(plsc.cummax(v), carry)
        ix_v[pl.ds(qoff + j * 16, 16)] = cm
        return jnp.max(cm)
    lax.fori_loop(0, POS_PW // 16, _cmax, c0)

    @pl.when(quarter == 0)
    def _():
        mel_v[...] = jnp.full((16,), total, jnp.int32)
        pltpu.sync_copy(mel_v, mel_hbm.at[row])

    obase = row * ML + qoff
    zeros16 = jnp.zeros((16,), jnp.float32)

    nvalids = [jnp.clip(total - (qoff + c * CHUNK), 0, CHUNK)
               for c in range(NCH)]

    def _window(c):
        lo = ix_v[pl.ds(qoff + c * CHUNK, 16)][0]
        return jnp.minimum((row * L + lo) // 8, B * L // 8 - SRCG)

    def _stage(c, sg):
        @pl.when(nvalids[c] > 0)
        def _():
            pltpu.async_copy(x_hbm.at[pl.ds(sg, SRCG)],
                             sbufs[c % NB], gsems[c % NB])

    def _stage_wait(c, sg):
        @pl.when(nvalids[c] > 0)
        def _():
            pltpu.make_async_copy(x_hbm.at[pl.ds(sg, SRCG)],
                                  sbufs[c % NB], gsems[c % NB]).wait()

    def _expand(c, k, sg):
        sbuf, obuf = sbufs[k], obufs[k]
        base_p = qoff + c * CHUNK
        nvalid = nvalids[c]

        def _copy_row(r, _):
            t = ix_v[pl.ds(base_p + r, 16)][0]
            off = row * L + t - sg * 8

            @pl.when(off < SRCG * 8)
            def _():
                for j in range(D // 16):
                    obuf[r // 8, r % 8, pl.ds(j * 16, 16)] = (
                        sbuf[off // 8, off % 8, pl.ds(j * 16, 16)])

            @pl.when(off >= SRCG * 8)
            def _():
                g = (row * L + t) // 8
                pltpu.sync_copy(x_hbm.at[pl.ds(g, 1)], fbuf)
                for j in range(D // 16):
                    obuf[r // 8, r % 8, pl.ds(j * 16, 16)] = (
                        fbuf[0, (row * L + t) % 8, pl.ds(j * 16, 16)])
            return 0
        lax.fori_loop(0, nvalid, _copy_row, 0)

        def _zero_row(r, _):
            for j in range(D // 16):
                obuf[r // 8, r % 8, pl.ds(j * 16, 16)] = zeros16
            return 0
        lax.fori_loop(nvalid, CHUNK, _zero_row, 0)

    starts = [None] * NCH
    starts[0] = _window(0)
    _stage(0, starts[0])

    def _zfill(r, _):
        for j in range(D // 16):
            zbuf[r // 8, r % 8, pl.ds(j * 16, 16)] = zeros16
        return 0
    lax.fori_loop(0, CHUNK, _zfill, 0)

    for c in range(NCH):
        k = c % NB
        nv = nvalids[c]
        if c + 1 < NCH:
            if c + 1 >= NB:
                pltpu.make_async_copy(
                    obufs[(c + 1) % NB],
                    out_hbm.at[pl.ds((obase + (c + 1 - NB) * CHUNK) // 8,
                                     CHUNK // 8)],
                    wsems[(c + 1) % NB]).wait()
            starts[c + 1] = _window(c + 1)
            _stage(c + 1, starts[c + 1])
        _stage_wait(c, starts[c])

        @pl.when(nv > 0)
        def _(c=c, k=k):
            _expand(c, k, starts[c])
            pltpu.async_copy(
                obufs[k],
                out_hbm.at[pl.ds((obase + c * CHUNK) // 8, CHUNK // 8)],
                wsems[k])

        @pl.when(nv == 0)
        def _(c=c, k=k):
            pltpu.async_copy(
                zbuf,
                out_hbm.at[pl.ds((obase + c * CHUNK) // 8, CHUNK // 8)],
                wsems[k])
    for c in range(NCH - NB, NCH):
        pltpu.make_async_copy(
            obufs[c % NB],
            out_hbm.at[pl.ds((obase + c * CHUNK) // 8, CHUNK // 8)],
            wsems[c % NB]).wait()


_lr_call = functools.partial(
    pl.kernel,
    out_type=[
        jax.ShapeDtypeStruct((B * ML // 8, 8, D), jnp.float32),
        jax.ShapeDtypeStruct((B, 16), jnp.int32),
    ],
    mesh=plsc.VectorSubcoreMesh(core_axis_name="c", subcore_axis_name="s"),
    compiler_params=pltpu.CompilerParams(needs_layout_passes=False),
    scratch_types=[
        pltpu.VMEM((L,), jnp.int32),
        pltpu.VMEM((ML,), jnp.int32),
        pltpu.VMEM((ML + 16,), jnp.int32),
        pltpu.VMEM((16,), jnp.int32),
        [pltpu.VMEM((SRCG, 8, D), jnp.float32) for _ in range(NB)],
        [pltpu.VMEM((CHUNK // 8, 8, D), jnp.float32) for _ in range(NB)],
        pltpu.VMEM((CHUNK // 8, 8, D), jnp.float32),
        pltpu.VMEM((1, 8, D), jnp.float32),
        [pltpu.SemaphoreType.DMA for _ in range(NB)],
        [pltpu.SemaphoreType.DMA for _ in range(NB)],
        pltpu.SemaphoreType.DMA,
    ],
)(_lr_body)


def kernel(x, duration, max_len):
    out3, mel = _lr_call(x.reshape(B * L // 8, 8, D), duration)
    return out3.reshape(B, ML, D), mel[:, 0]


# speedup vs baseline: 12.6018x; 2.0741x over previous
import functools

import jax
import jax.numpy as jnp
from jax import lax
from jax.experimental import pallas as pl
from jax.experimental.pallas import tpu as pltpu
from jax.experimental.pallas import tpu_sc as plsc

B, L, D = 8, 512, 256
ML = 3584
NW = 32
WPR = NW // B
POS_PW = ML // WPR
CHUNK = 112
NCH = POS_PW // CHUNK
SRCG = 8
NB = 2


def _lr_body(x_hbm, dur_hbm, out_hbm, mel_hbm,
             d_v, s_v, ix_v, mel_v, sbufs, obufs, zbuf, fbuf,
             gsems, wsems, fsem):
    cid = lax.axis_index("c")
    sid = lax.axis_index("s")
    wid = cid * 16 + sid
    row = wid // WPR
    quarter = wid % WPR

    pltpu.sync_copy(dur_hbm.at[row], d_v)

    lane = lax.iota(jnp.int32, 16)
    qoff = quarter * POS_PW

    def _zero(j, _):
        s_v[pl.ds(qoff + j * 16, 16)] = jnp.zeros((16,), jnp.int32)
        return 0
    lax.fori_loop(0, POS_PW // 16, _zero, 0)

    def _scan(j, carry):
        tot, c0 = carry
        d = d_v[pl.ds(j * 16, 16)]
        cs = plsc.cumsum(d) + tot
        start = cs - d
        tok = j * 16 + lane
        m = (d > 0) & (start < ML)
        plsc.store_scatter(s_v, [start], tok, mask=m)
        c0 = c0 + plsc.all_reduce_population_count(cs <= qoff)[0]
        return jnp.max(cs), c0
    total, c0 = lax.fori_loop(0, L // 16, _scan,
                              (jnp.int32(0), jnp.int32(0)))

    def _cmax(j, carry):
        v = s_v[pl.ds(qoff + j * 16, 16)]
        cm = jnp.maximum(plsc.cummax(v), carry)
        ix_v[pl.ds(qoff + j * 16, 16)] = cm
        return jnp.max(cm)
    lax.fori_loop(0, POS_PW // 16, _cmax, c0)

    @pl.when(quarter == 0)
    def _():
        mel_v[...] = jnp.full((16,), total, jnp.int32)
        pltpu.sync_copy(mel_v, mel_hbm.at[row])

    obase = row * ML + qoff
    zeros16 = jnp.zeros((16,), jnp.float32)
    MAXO = SRCG * 8 - 1

    def _nvalid(c):
        return jnp.clip(total - (qoff + c * CHUNK), 0, CHUNK)

    def _window(c):
        lo = ix_v[pl.ds(qoff + c * CHUNK, 16)][0]
        return jnp.minimum((row * L + lo) // 8, B * L // 8 - SRCG)

    def _stage(c, k):
        @pl.when((c < NCH) & (_nvalid(c) > 0))
        def _():
            pltpu.async_copy(x_hbm.at[pl.ds(_window(c), SRCG)],
                             sbufs[k], gsems[k])

    def _stage_wait(c, k):
        @pl.when((c < NCH) & (_nvalid(c) > 0))
        def _():
            pltpu.make_async_copy(x_hbm.at[pl.ds(_window(c), SRCG)],
                                  sbufs[k], gsems[k]).wait()

    def _wdrain(c, k):
        pltpu.make_async_copy(
            obufs[k],
            out_hbm.at[pl.ds((obase + c * CHUNK) // 8, CHUNK // 8)],
            wsems[k]).wait()

    def _expand(c, k, sg):
        sbuf, obuf = sbufs[k], obufs[k]
        base_p = qoff + c * CHUNK
        nv = _nvalid(c)

        def _rowcopy(r, off):
            vals = [sbuf[off // 8, off % 8, pl.ds(j * 16, 16)]
                    for j in range(D // 16)]
            for j in range(D // 16):
                obuf[r // 8, r % 8, pl.ds(j * 16, 16)] = vals[j]

        def _copy16(g, _):
            tv = ix_v[pl.ds(base_p + g * 16, 16)]
            offv = jnp.minimum(row * L + tv - sg * 8, MAXO)
            for r8 in range(16):
                _rowcopy(g * 16 + r8, offv[r8])
            return 0
        lax.fori_loop(0, nv // 16, _copy16, 0)

        def _copy1(r, _):
            t = ix_v[pl.ds(base_p + r, 16)][0]
            _rowcopy(r, jnp.minimum(row * L + t - sg * 8, MAXO))
            return 0
        lax.fori_loop((nv // 16) * 16, nv, _copy1, 0)

        last_t = ix_v[pl.ds(base_p + nv - 1, 16)][0]

        @pl.when(row * L + last_t - sg * 8 >= SRCG * 8)
        def _():
            def _fix(r, _):
                t = ix_v[pl.ds(base_p + r, 16)][0]

                @pl.when(row * L + t - sg * 8 >= SRCG * 8)
                def _():
                    g = (row * L + t) // 8
                    pltpu.sync_copy(x_hbm.at[pl.ds(g, 1)], fbuf)
                    for j in range(D // 16):
                        obuf[r // 8, r % 8, pl.ds(j * 16, 16)] = (
                            fbuf[0, (row * L + t) % 8, pl.ds(j * 16, 16)])
                return 0
            lax.fori_loop(0, nv, _fix, 0)

        def _zero1(r, _):
            for j in range(D // 16):
                obuf[r // 8, r % 8, pl.ds(j * 16, 16)] = zeros16
            return 0

        def _zero16(g, _):
            for r8 in range(16):
                og = g * 2 + r8 // 8
                for j in range(D // 16):
                    obuf[og, r8 % 8, pl.ds(j * 16, 16)] = zeros16
            return 0
        zfrom = (nv + 15) // 16
        lax.fori_loop(nv, jnp.minimum(zfrom * 16, CHUNK), _zero1, 0)
        lax.fori_loop(zfrom, CHUNK // 16, _zero16, 0)

    def _do_chunk(c, k):
        nv = _nvalid(c)

        @pl.when((c < NCH) & (nv > 0))
        def _():
            _expand(c, k, _window(c))
            pltpu.async_copy(
                obufs[k],
                out_hbm.at[pl.ds((obase + c * CHUNK) // 8, CHUNK // 8)],
                wsems[k])

        @pl.when((c < NCH) & (nv == 0))
        def _():
            pltpu.async_copy(
                zbuf,
                out_hbm.at[pl.ds((obase + c * CHUNK) // 8, CHUNK // 8)],
                wsems[k])

    _stage(jnp.int32(0), 0)

    def _zfill(r, _):
        for j in range(D // 16):
            zbuf[r // 8, r % 8, pl.ds(j * 16, 16)] = zeros16
        return 0
    lax.fori_loop(0, CHUNK, _zfill, 0)

    def _pair(cc, _):
        c = cc * 2
        _stage(c + 1, 1)

        @pl.when(cc > 0)
        def _():
            _wdrain(c - 2, 0)
        _stage_wait(c, 0)
        _do_chunk(c, 0)

        _stage(c + 2, 0)

        @pl.when(cc > 0)
        def _():
            _wdrain(c - 1, 1)
        _stage_wait(c + 1, 1)
        _do_chunk(c + 1, 1)
        return 0
    lax.fori_loop(0, NCH // 2, _pair, 0)
    _wdrain(NCH - 2, 0)
    _wdrain(NCH - 1, 1)


_lr_call = functools.partial(
    pl.kernel,
    out_type=[
        jax.ShapeDtypeStruct((B * ML // 8, 8, D), jnp.float32),
        jax.ShapeDtypeStruct((B, 16), jnp.int32),
    ],
    mesh=plsc.VectorSubcoreMesh(core_axis_name="c", subcore_axis_name="s"),
    compiler_params=pltpu.CompilerParams(needs_layout_passes=False),
    scratch_types=[
        pltpu.VMEM((L,), jnp.int32),
        pltpu.VMEM((ML,), jnp.int32),
        pltpu.VMEM((ML + 16,), jnp.int32),
        pltpu.VMEM((16,), jnp.int32),
        [pltpu.VMEM((SRCG, 8, D), jnp.float32) for _ in range(NB)],
        [pltpu.VMEM((CHUNK // 8, 8, D), jnp.float32) for _ in range(NB)],
        pltpu.VMEM((CHUNK // 8, 8, D), jnp.float32),
        pltpu.VMEM((1, 8, D), jnp.float32),
        [pltpu.SemaphoreType.DMA for _ in range(NB)],
        [pltpu.SemaphoreType.DMA for _ in range(NB)],
        pltpu.SemaphoreType.DMA,
    ],
)(_lr_body)


def kernel(x, duration, max_len):
    out3, mel = _lr_call(x.reshape(B * L // 8, 8, D), duration)
    return out3.reshape(B, ML, D), mel[:, 0]
